# baseline (speedup 1.0000x reference)
import functools

import jax
import jax.numpy as jnp
import numpy as np
from jax import lax
from jax.experimental import pallas as pl
from jax.experimental.pallas import tpu as pltpu
from jax.experimental.pallas import tpu_sc as plsc

B = 4096
F = 26
V = 100000
D = 64
NF = F + 1
R = B * F

NC = 2
NS = 16
NW = NC * NS
LANES = 16
RPW = R // NW
CH = 128
NCHUNK = RPW // CH
NBUF = 4


def _make_sc_gather():
    mesh = plsc.VectorSubcoreMesh(core_axis_name="c", subcore_axis_name="s")

    @functools.partial(
        pl.kernel,
        out_type=jax.ShapeDtypeStruct((R, 2 * D), jnp.float32),
        mesh=mesh,
        scratch_types=[
            pltpu.VMEM((RPW,), jnp.int32),
            pltpu.VMEM((NCHUNK, CH), jnp.int32),
            pltpu.VMEM((NBUF, CH, 2 * D), jnp.float32),
            pltpu.SemaphoreType.DMA((NBUF,)),
        ],
    )
    def sc_gather(tab_hbm, idx_hbm, out_hbm, idx_raw, idx_adj, rows, sems):
        wid = lax.axis_index("s") * NC + lax.axis_index("c")
        base = wid * RPW
        pltpu.sync_copy(idx_hbm.at[pl.ds(base, RPW)], idx_raw)

        @pl.loop(0, NCHUNK)
        def _offsets(c):
            for j in range(CH // LANES):
                p0 = base + c * CH + j * LANES
                p_vec = jax.lax.iota(jnp.int32, LANES) + p0
                off = (p_vec % F) * (V // 2)
                raw = idx_raw[pl.ds(c * CH + j * LANES, LANES)]
                idx_adj[c, pl.ds(j * LANES, LANES)] = (
                    jax.lax.shift_right_logical(raw, 1) + off
                )

        def _gather(c, b):
            pltpu.async_copy(tab_hbm.at[idx_adj.at[c]], rows.at[b], sems.at[b])

        def _drain(c, b):
            pltpu.make_async_copy(tab_hbm.at[idx_adj.at[c]], rows.at[b], sems.at[b]).wait()
            pltpu.sync_copy(rows.at[b], out_hbm.at[pl.ds(base + c * CH, CH)])

        for b in range(NBUF):
            _gather(b, b)

        @pl.loop(0, NCHUNK // NBUF)
        def _pipe(g):
            for b in range(NBUF):
                c = g * NBUF + b
                _drain(c, b)
                nxt = c + NBUF

                @pl.when(nxt < NCHUNK)
                def _():
                    _gather(nxt, b)

        rem = NCHUNK % NBUF
        for b in range(rem):
            _drain(NCHUNK - rem + b, b)

    return sc_gather


def _tc_body(
    x_ref,
    emb_ref,
    par_ref,
    dw0,
    db0,
    dw1,
    db1,
    dw2,
    db2,
    ow0d,
    ow0e,
    ob0,
    ow1,
    ob1,
    ow2,
    ob2,
    ow3,
    ob3,
    out_ref,
):
    f32 = jnp.float32
    x = x_ref[...]
    h = jnp.maximum(jnp.dot(x, dw0[...], preferred_element_type=f32) + db0[...], 0.0)
    h = jnp.maximum(jnp.dot(h, dw1[...], preferred_element_type=f32) + db1[...], 0.0)
    ed = jnp.maximum(jnp.dot(h, dw2[...], preferred_element_type=f32) + db2[...], 0.0)
    bB = x.shape[0]
    emb = emb_ref[...].reshape(bB, F, 2 * D)
    par = par_ref[...]
    sel = jnp.where(
        (par[:, :, None] & 1) == 1, emb[:, :, D:], emb[:, :, :D]
    )
    c3 = jnp.concatenate([ed[:, None, :], sel], axis=1)
    g3 = jax.lax.dot_general(
        c3, c3, (((2,), (2,)), ((0,), (0,))), preferred_element_type=f32
    )
    g = g3.reshape(bB, NF * NF)
    y = jnp.dot(ed, ow0d[...], preferred_element_type=f32)
    y = y + jnp.dot(g, ow0e[...], preferred_element_type=f32)
    y = jnp.maximum(y + ob0[...], 0.0)
    y = jnp.maximum(jnp.dot(y, ow1[...], preferred_element_type=f32) + ob1[...], 0.0)
    y = jnp.maximum(jnp.dot(y, ow2[...], preferred_element_type=f32) + ob2[...], 0.0)
    out_ref[...] = jnp.dot(y, ow3[...], preferred_element_type=f32) + ob3[...]


def _full(shape):
    return pl.BlockSpec(shape, lambda i: (0, 0))


def _tc_forward(x, emb2, par, weights, bB):
    nb = B // bB
    in_specs = [
        pl.BlockSpec((bB, x.shape[1]), lambda i: (i, 0)),
        pl.BlockSpec((bB, F * 2 * D), lambda i: (i, 0)),
        pl.BlockSpec((bB, F), lambda i: (i, 0)),
    ] + [_full(w.shape) for w in weights]
    return pl.pallas_call(
        _tc_body,
        grid=(nb,),
        in_specs=in_specs,
        out_specs=pl.BlockSpec((bB, 1), lambda i: (i, 0)),
        out_shape=jax.ShapeDtypeStruct((B, 1), jnp.float32),
    )(x, emb2, par, *weights)


def kernel(dense_features, sparse_indices, tables, dense_params, over_params):
    tables_flat = tables.reshape(F * V // 2, 2 * D)
    sparse_indices = sparse_indices.astype(jnp.int32)
    idx1d = sparse_indices.reshape(R)
    emb = _make_sc_gather()(tables_flat, idx1d)
    emb2 = emb.reshape(B, F * 2 * D)

    (dw0, db0), (dw1, db1), (dw2, db2) = dense_params
    (ow0, ob0), (ow1, ob1), (ow2, ob2), (ow3, ob3) = over_params
    ti, tj = np.triu_indices(NF, k=1)
    ow0d = ow0[:D]
    ow0e = jnp.zeros((NF * NF, ow0.shape[1]), jnp.float32)
    ow0e = ow0e.at[ti * NF + tj].set(ow0[D:])
    weights = [
        dw0,
        db0.reshape(1, -1),
        dw1,
        db1.reshape(1, -1),
        dw2,
        db2.reshape(1, -1),
        ow0d,
        ow0e,
        ob0.reshape(1, -1),
        ow1,
        ob1.reshape(1, -1),
        ow2,
        ob2.reshape(1, -1),
        ow3,
        ob3.reshape(1, -1),
    ]
    return _tc_forward(dense_features, emb2, sparse_indices, weights, bB=512)


# speedup vs baseline: 1.0413x; 1.0413x over previous
import functools

import jax
import jax.numpy as jnp
import numpy as np
from jax import lax
from jax.experimental import pallas as pl
from jax.experimental.pallas import tpu as pltpu
from jax.experimental.pallas import tpu_sc as plsc

B = 4096
F = 26
V = 100000
D = 64
NF = F + 1
R = B * F

NC = 2
NS = 16
NW = NC * NS
LANES = 16
RPW = R // NW
CH = 128
NCHUNK = RPW // CH
NBUF = 4


def _make_sc_gather():
    mesh = plsc.VectorSubcoreMesh(core_axis_name="c", subcore_axis_name="s")

    @functools.partial(
        pl.kernel,
        out_type=jax.ShapeDtypeStruct((R, 2 * D), jnp.float32),
        mesh=mesh,
        scratch_types=[
            pltpu.VMEM((RPW,), jnp.int32),
            pltpu.VMEM((NCHUNK, CH), jnp.int32),
            pltpu.VMEM((NBUF, CH, 2 * D), jnp.float32),
            pltpu.SemaphoreType.DMA((NBUF,)),
        ],
    )
    def sc_gather(tab_hbm, idx_hbm, out_hbm, idx_raw, idx_adj, rows, sems):
        wid = lax.axis_index("s") * NC + lax.axis_index("c")
        base = wid * RPW
        pltpu.sync_copy(idx_hbm.at[pl.ds(base, RPW)], idx_raw)

        @pl.loop(0, NCHUNK)
        def _offsets(c):
            for j in range(CH // LANES):
                p0 = base + c * CH + j * LANES
                p_vec = jax.lax.iota(jnp.int32, LANES) + p0
                off = (p_vec % F) * (V // 2)
                raw = idx_raw[pl.ds(c * CH + j * LANES, LANES)]
                idx_adj[c, pl.ds(j * LANES, LANES)] = (
                    jax.lax.shift_right_logical(raw, 1) + off
                )

        def _gather(c, b):
            pltpu.async_copy(tab_hbm.at[idx_adj.at[c]], rows.at[b], sems.at[b])

        def _drain(c, b):
            pltpu.make_async_copy(tab_hbm.at[idx_adj.at[c]], rows.at[b], sems.at[b]).wait()
            pltpu.sync_copy(rows.at[b], out_hbm.at[pl.ds(base + c * CH, CH)])

        for b in range(NBUF):
            _gather(b, b)

        @pl.loop(0, NCHUNK // NBUF)
        def _pipe(g):
            for b in range(NBUF):
                c = g * NBUF + b
                _drain(c, b)
                nxt = c + NBUF

                @pl.when(nxt < NCHUNK)
                def _():
                    _gather(nxt, b)

        rem = NCHUNK % NBUF
        for b in range(rem):
            _drain(NCHUNK - rem + b, b)

    return sc_gather


def _tc_body(
    x_ref,
    emb_ref,
    par_ref,
    dw0,
    db0,
    dw1,
    db1,
    dw2,
    db2,
    ow0d,
    ow0e,
    ob0,
    ow1,
    ob1,
    ow2,
    ob2,
    ow3,
    ob3,
    out_ref,
):
    f32 = jnp.float32
    x = x_ref[...]
    h = jnp.maximum(jnp.dot(x, dw0[...], preferred_element_type=f32) + db0[...], 0.0)
    h = jnp.maximum(jnp.dot(h, dw1[...], preferred_element_type=f32) + db1[...], 0.0)
    ed = jnp.maximum(jnp.dot(h, dw2[...], preferred_element_type=f32) + db2[...], 0.0)
    bB = x.shape[0]
    emb = emb_ref[...].reshape(bB, F, 2 * D)
    par = par_ref[...]
    sel = jnp.where(
        (par[:, :, None] & 1) == 1, emb[:, :, D:], emb[:, :, :D]
    )
    c3 = jnp.concatenate([ed[:, None, :], sel], axis=1)
    g3 = jax.lax.dot_general(
        c3, c3, (((2,), (2,)), ((0,), (0,))), preferred_element_type=f32
    )
    g = g3.reshape(bB, NF * NF)
    y = jnp.dot(ed, ow0d[...], preferred_element_type=f32)
    y = y + jnp.dot(g, ow0e[...], preferred_element_type=f32)
    y = jnp.maximum(y + ob0[...], 0.0)
    y = jnp.maximum(jnp.dot(y, ow1[...], preferred_element_type=f32) + ob1[...], 0.0)
    y = jnp.maximum(jnp.dot(y, ow2[...], preferred_element_type=f32) + ob2[...], 0.0)
    out_ref[...] = jnp.dot(y, ow3[...], preferred_element_type=f32) + ob3[...]


def _full(shape):
    return pl.BlockSpec(shape, lambda i: (0, 0))


def _tc_forward(x, emb2, par, weights, bB):
    nb = B // bB
    in_specs = [
        pl.BlockSpec((bB, x.shape[1]), lambda i: (i, 0)),
        pl.BlockSpec((bB * F, 2 * D), lambda i: (i, 0)),
        pl.BlockSpec((bB, F), lambda i: (i, 0)),
    ] + [_full(w.shape) for w in weights]
    return pl.pallas_call(
        _tc_body,
        grid=(nb,),
        in_specs=in_specs,
        out_specs=pl.BlockSpec((bB, 1), lambda i: (i, 0)),
        out_shape=jax.ShapeDtypeStruct((B, 1), jnp.float32),
    )(x, emb2, par, *weights)


def kernel(dense_features, sparse_indices, tables, dense_params, over_params):
    tables_flat = tables.reshape(F * V // 2, 2 * D)
    sparse_indices = sparse_indices.astype(jnp.int32)
    idx1d = sparse_indices.reshape(R)
    emb2 = _make_sc_gather()(tables_flat, idx1d)

    (dw0, db0), (dw1, db1), (dw2, db2) = dense_params
    (ow0, ob0), (ow1, ob1), (ow2, ob2), (ow3, ob3) = over_params
    ti, tj = np.triu_indices(NF, k=1)
    ow0d = ow0[:D]
    ow0e = jnp.zeros((NF * NF, ow0.shape[1]), jnp.float32)
    ow0e = ow0e.at[ti * NF + tj].set(ow0[D:])
    weights = [
        dw0,
        db0.reshape(1, -1),
        dw1,
        db1.reshape(1, -1),
        dw2,
        db2.reshape(1, -1),
        ow0d,
        ow0e,
        ob0.reshape(1, -1),
        ow1,
        ob1.reshape(1, -1),
        ow2,
        ob2.reshape(1, -1),
        ow3,
        ob3.reshape(1, -1),
    ]
    return _tc_forward(dense_features, emb2, sparse_indices, weights, bB=512)


# speedup vs baseline: 1.0444x; 1.0029x over previous
import functools

import jax
import jax.numpy as jnp
import numpy as np
from jax import lax
from jax.experimental import pallas as pl
from jax.experimental.pallas import tpu as pltpu
from jax.experimental.pallas import tpu_sc as plsc

B = 4096
F = 26
V = 100000
D = 64
NF = F + 1
R = B * F

NC = 2
NS = 16
NW = NC * NS
LANES = 16
RPW = R // NW
CH = 128
NCHUNK = RPW // CH
NBUF = 4


def _make_sc_gather():
    mesh = plsc.VectorSubcoreMesh(core_axis_name="c", subcore_axis_name="s")

    @functools.partial(
        pl.kernel,
        out_type=jax.ShapeDtypeStruct((R, D), jnp.float32),
        mesh=mesh,
        scratch_types=[
            pltpu.VMEM((RPW,), jnp.int32),
            pltpu.VMEM((NCHUNK, CH), jnp.int32),
            pltpu.VMEM((NBUF, CH, D), jnp.float32),
            pltpu.SemaphoreType.DMA((NBUF,)),
        ],
        compiler_params=pltpu.CompilerParams(use_tc_tiling_on_sc=False),
    )
    def sc_gather(tab_hbm, idx_hbm, out_hbm, idx_raw, idx_adj, rows, sems):
        wid = lax.axis_index("s") * NC + lax.axis_index("c")
        base = wid * RPW
        pltpu.sync_copy(idx_hbm.at[pl.ds(base, RPW)], idx_raw)

        @pl.loop(0, NCHUNK)
        def _offsets(c):
            for j in range(CH // LANES):
                p0 = base + c * CH + j * LANES
                p_vec = jax.lax.iota(jnp.int32, LANES) + p0
                off = (p_vec % F) * V
                idx_adj[c, pl.ds(j * LANES, LANES)] = (
                    idx_raw[pl.ds(c * CH + j * LANES, LANES)] + off
                )

        def _gather(c, b):
            pltpu.async_copy(tab_hbm.at[idx_adj.at[c]], rows.at[b], sems.at[b])

        def _drain(c, b):
            pltpu.make_async_copy(tab_hbm.at[idx_adj.at[c]], rows.at[b], sems.at[b]).wait()
            pltpu.sync_copy(rows.at[b], out_hbm.at[pl.ds(base + c * CH, CH)])

        for b in range(NBUF):
            _gather(b, b)

        @pl.loop(0, NCHUNK // NBUF)
        def _pipe(g):
            for b in range(NBUF):
                c = g * NBUF + b
                _drain(c, b)
                nxt = c + NBUF

                @pl.when(nxt < NCHUNK)
                def _():
                    _gather(nxt, b)

        rem = NCHUNK % NBUF
        for b in range(rem):
            _drain(NCHUNK - rem + b, b)

    return sc_gather


def _tc_body(
    x_ref,
    emb_ref,
    dw0,
    db0,
    dw1,
    db1,
    dw2,
    db2,
    ow0d,
    ow0e,
    ob0,
    ow1,
    ob1,
    ow2,
    ob2,
    ow3,
    ob3,
    out_ref,
):
    f32 = jnp.float32
    x = x_ref[...]
    h = jnp.maximum(jnp.dot(x, dw0[...], preferred_element_type=f32) + db0[...], 0.0)
    h = jnp.maximum(jnp.dot(h, dw1[...], preferred_element_type=f32) + db1[...], 0.0)
    ed = jnp.maximum(jnp.dot(h, dw2[...], preferred_element_type=f32) + db2[...], 0.0)
    bB = x.shape[0]
    emb = emb_ref[...].reshape(bB, F, D)
    c3 = jnp.concatenate([ed[:, None, :], emb], axis=1)
    g3 = jax.lax.dot_general(
        c3, c3, (((2,), (2,)), ((0,), (0,))), preferred_element_type=f32
    )
    g = g3.reshape(bB, NF * NF)
    y = jnp.dot(ed, ow0d[...], preferred_element_type=f32)
    y = y + jnp.dot(g, ow0e[...], preferred_element_type=f32)
    y = jnp.maximum(y + ob0[...], 0.0)
    y = jnp.maximum(jnp.dot(y, ow1[...], preferred_element_type=f32) + ob1[...], 0.0)
    y = jnp.maximum(jnp.dot(y, ow2[...], preferred_element_type=f32) + ob2[...], 0.0)
    out_ref[...] = jnp.dot(y, ow3[...], preferred_element_type=f32) + ob3[...]


def _full(shape):
    return pl.BlockSpec(shape, lambda i: (0, 0))


def _tc_forward(x, emb2, weights, bB):
    nb = B // bB
    in_specs = [
        pl.BlockSpec((bB, x.shape[1]), lambda i: (i, 0)),
        pl.BlockSpec((bB * F, D), lambda i: (i, 0)),
    ] + [_full(w.shape) for w in weights]
    return pl.pallas_call(
        _tc_body,
        grid=(nb,),
        in_specs=in_specs,
        out_specs=pl.BlockSpec((bB, 1), lambda i: (i, 0)),
        out_shape=jax.ShapeDtypeStruct((B, 1), jnp.float32),
    )(x, emb2, *weights)


def kernel(dense_features, sparse_indices, tables, dense_params, over_params):
    tables_flat = tables.reshape(F * V, D)
    sparse_indices = sparse_indices.astype(jnp.int32)
    idx1d = sparse_indices.reshape(R)
    emb2 = _make_sc_gather()(tables_flat, idx1d)

    (dw0, db0), (dw1, db1), (dw2, db2) = dense_params
    (ow0, ob0), (ow1, ob1), (ow2, ob2), (ow3, ob3) = over_params
    ti, tj = np.triu_indices(NF, k=1)
    ow0d = ow0[:D]
    ow0e = jnp.zeros((NF * NF, ow0.shape[1]), jnp.float32)
    ow0e = ow0e.at[ti * NF + tj].set(ow0[D:])
    weights = [
        dw0,
        db0.reshape(1, -1),
        dw1,
        db1.reshape(1, -1),
        dw2,
        db2.reshape(1, -1),
        ow0d,
        ow0e,
        ob0.reshape(1, -1),
        ow1,
        ob1.reshape(1, -1),
        ow2,
        ob2.reshape(1, -1),
        ow3,
        ob3.reshape(1, -1),
    ]
    return _tc_forward(dense_features, emb2, weights, bB=512)


# speedup vs baseline: 1.0752x; 1.0295x over previous
import functools

import jax
import jax.numpy as jnp
import numpy as np
from jax import lax
from jax.experimental import pallas as pl
from jax.experimental.pallas import tpu as pltpu
from jax.experimental.pallas import tpu_sc as plsc

B = 4096
F = 26
V = 100000
D = 64
NF = F + 1
R = B * F

NC = 2
NS = 16
NW = NC * NS
LANES = 16
RPW = R // NW
CH = 128
NCHUNK = RPW // CH
NBUF = 4


def _make_sc_gather():
    mesh = plsc.VectorSubcoreMesh(core_axis_name="c", subcore_axis_name="s")

    @functools.partial(
        pl.kernel,
        out_type=jax.ShapeDtypeStruct((R, 2 * D), jnp.float32),
        mesh=mesh,
        scratch_types=[
            pltpu.VMEM((RPW,), jnp.int32),
            pltpu.VMEM((NCHUNK, CH), jnp.int32),
            pltpu.VMEM((NBUF, CH, D), jnp.float32),
            pltpu.SemaphoreType.DMA((NBUF,)),
        ],
        compiler_params=pltpu.CompilerParams(use_tc_tiling_on_sc=False),
    )
    def sc_gather(tab_hbm, idx_hbm, out_hbm, idx_raw, idx_adj, rows, sems):
        wid = lax.axis_index("s") * NC + lax.axis_index("c")
        base = wid * RPW
        pltpu.sync_copy(idx_hbm.at[pl.ds(base, RPW)], idx_raw)

        @pl.loop(0, NCHUNK)
        def _offsets(c):
            for j in range(CH // LANES):
                p0 = base + c * CH + j * LANES
                p_vec = jax.lax.iota(jnp.int32, LANES) + p0
                off = (p_vec % F) * V
                idx_adj[c, pl.ds(j * LANES, LANES)] = (
                    idx_raw[pl.ds(c * CH + j * LANES, LANES)] + off
                )

        def _gather(c, b):
            pltpu.async_copy(tab_hbm.at[idx_adj.at[c]], rows.at[b], sems.at[b])

        def _drain(c, b):
            pltpu.make_async_copy(tab_hbm.at[idx_adj.at[c]], rows.at[b], sems.at[b]).wait()
            pltpu.sync_copy(
                rows.at[b], out_hbm.at[pl.ds(base + c * CH, CH), pl.ds(0, D)]
            )

        for b in range(NBUF):
            _gather(b, b)

        @pl.loop(0, NCHUNK // NBUF)
        def _pipe(g):
            for b in range(NBUF):
                c = g * NBUF + b
                _drain(c, b)
                nxt = c + NBUF

                @pl.when(nxt < NCHUNK)
                def _():
                    _gather(nxt, b)

        rem = NCHUNK % NBUF
        for b in range(rem):
            _drain(NCHUNK - rem + b, b)

    return sc_gather


def _tc_body(
    x_ref,
    emb_ref,
    dw0,
    db0,
    dw1,
    db1,
    dw2,
    db2,
    ow0d,
    ow0e,
    ob0,
    ow1,
    ob1,
    ow2,
    ob2,
    ow3,
    ob3,
    out_ref,
):
    f32 = jnp.float32
    x = x_ref[...]
    h = jnp.maximum(jnp.dot(x, dw0[...], preferred_element_type=f32) + db0[...], 0.0)
    h = jnp.maximum(jnp.dot(h, dw1[...], preferred_element_type=f32) + db1[...], 0.0)
    ed = jnp.maximum(jnp.dot(h, dw2[...], preferred_element_type=f32) + db2[...], 0.0)
    bB = x.shape[0]
    emb = emb_ref[...].reshape(bB, F, 2 * D)[:, :, :D]
    c3 = jnp.concatenate([ed[:, None, :], emb], axis=1)
    g3 = jax.lax.dot_general(
        c3, c3, (((2,), (2,)), ((0,), (0,))), preferred_element_type=f32
    )
    g = g3.reshape(bB, NF * NF)
    y = jnp.dot(ed, ow0d[...], preferred_element_type=f32)
    y = y + jnp.dot(g, ow0e[...], preferred_element_type=f32)
    y = jnp.maximum(y + ob0[...], 0.0)
    y = jnp.maximum(jnp.dot(y, ow1[...], preferred_element_type=f32) + ob1[...], 0.0)
    y = jnp.maximum(jnp.dot(y, ow2[...], preferred_element_type=f32) + ob2[...], 0.0)
    out_ref[...] = jnp.dot(y, ow3[...], preferred_element_type=f32) + ob3[...]


def _full(shape):
    return pl.BlockSpec(shape, lambda i: (0, 0))


def _tc_forward(x, emb2, weights, bB):
    nb = B // bB
    in_specs = [
        pl.BlockSpec((bB, x.shape[1]), lambda i: (i, 0)),
        pl.BlockSpec((bB * F, 2 * D), lambda i: (i, 0)),
    ] + [_full(w.shape) for w in weights]
    return pl.pallas_call(
        _tc_body,
        grid=(nb,),
        in_specs=in_specs,
        out_specs=pl.BlockSpec((bB, 1), lambda i: (i, 0)),
        out_shape=jax.ShapeDtypeStruct((B, 1), jnp.float32),
    )(x, emb2, *weights)


def kernel(dense_features, sparse_indices, tables, dense_params, over_params):
    tables_flat = tables.reshape(F * V, D)
    sparse_indices = sparse_indices.astype(jnp.int32)
    idx1d = sparse_indices.reshape(R)
    emb2 = _make_sc_gather()(tables_flat, idx1d)

    (dw0, db0), (dw1, db1), (dw2, db2) = dense_params
    (ow0, ob0), (ow1, ob1), (ow2, ob2), (ow3, ob3) = over_params
    ti, tj = np.triu_indices(NF, k=1)
    ow0d = ow0[:D]
    ow0e = jnp.zeros((NF * NF, ow0.shape[1]), jnp.float32)
    ow0e = ow0e.at[ti * NF + tj].set(ow0[D:])
    weights = [
        dw0,
        db0.reshape(1, -1),
        dw1,
        db1.reshape(1, -1),
        dw2,
        db2.reshape(1, -1),
        ow0d,
        ow0e,
        ob0.reshape(1, -1),
        ow1,
        ob1.reshape(1, -1),
        ow2,
        ob2.reshape(1, -1),
        ow3,
        ob3.reshape(1, -1),
    ]
    return _tc_forward(dense_features, emb2, weights, bB=512)
